# baseline (speedup 1.0000x reference)
import functools

import jax
import jax.numpy as jnp
from jax import lax
from jax.experimental import pallas as pl
from jax.experimental.pallas import tpu as pltpu
from jax.experimental.pallas import tpu_sc as plsc

_NC = 2
_NS = 16
_NW = _NC * _NS
_L = 16
_CHUNK = 100
_NBUF = 4
_CBREL = 2048


def _sc_pooled_sum(x2, table):
  twoB, chunk = x2.shape
  assert chunk == _CHUNK
  B = twoB // 2
  D = 2 * table.shape[1]
  assert D % (2 * _L) == 0 and B % _NW == 0
  b_per_w = B // _NW
  c_per_w = 2 * b_per_w
  nd = D // _L

  mesh = plsc.VectorSubcoreMesh(
      core_axis_name="c", subcore_axis_name="s",
      num_cores=_NC, num_subcores=_NS)

  @functools.partial(
      pl.kernel,
      out_type=jax.ShapeDtypeStruct((B, D), jnp.float32),
      mesh=mesh,
      scratch_types=[
          pltpu.VMEM((c_per_w, _CHUNK), jnp.int32),
          pltpu.VMEM((_NBUF, _CHUNK, D // 2), jnp.int32),
          pltpu.VMEM((b_per_w, D), jnp.float32),
      ] + [pltpu.SemaphoreType.DMA] * _NBUF,
      compiler_params=pltpu.CompilerParams(use_tc_tiling_on_sc=False),
  )
  def k(x2_hbm, table_hbm, out_hbm, idx_v, rows_v, out_v, *sems):
    wid = lax.axis_index("s") * _NC + lax.axis_index("c")
    cbase = wid * c_per_w
    bbase = wid * b_per_w

    pltpu.sync_copy(x2_hbm.at[pl.ds(cbase, c_per_w)], idx_v)

    for k0 in range(_NBUF):
      pltpu.async_copy(table_hbm.at[idx_v.at[k0]], rows_v.at[k0], sems[k0])

    def accum_chunk(buf, accs):
      def body(r4, accs):
        accs = list(accs)
        for u in range(4):
          r = r4 * 4 + u
          for h in range(nd // 2):
            w = rows_v[buf, r, h * _L:(h + 1) * _L]
            a = lax.bitcast_convert_type(w << 16, jnp.float32)
            b = lax.bitcast_convert_type(
                w & jnp.int32(-65536), jnp.float32)
            accs[h] = accs[h] + a
            accs[h + 2] = accs[h + 2] + b
        return tuple(accs)
      return lax.fori_loop(0, _CHUNK // 4, body, accs)

    def pair_body(p, carry):
      for half in range(2):
        i = 2 * p + half
        accs = tuple(jnp.zeros((_L,), jnp.float32) for _ in range(nd))
        for k1 in (2 * half, 2 * half + 1):
          c = 4 * p + k1
          pltpu.make_async_copy(
              table_hbm.at[idx_v.at[c]], rows_v.at[k1], sems[k1]).wait()
          accs = accum_chunk(k1, accs)

          @pl.when(c + _NBUF < c_per_w)
          def _():
            pltpu.async_copy(
                table_hbm.at[idx_v.at[c + _NBUF]], rows_v.at[k1], sems[k1])

        for d in range(nd):
          out_v[i, d * _L:(d + 1) * _L] = accs[d]
      return carry

    lax.fori_loop(0, b_per_w // 2, pair_body, 0)
    pltpu.sync_copy(out_v, out_hbm.at[pl.ds(bbase, b_per_w)])

  return k(x2, table)


def _rne_bf16_bits(x):
  bits = lax.bitcast_convert_type(x, jnp.int32)
  return bits + jnp.int32(0x7FFF) + ((bits >> 16) & jnp.int32(1))


def _tc_relayout(tT):
  D, V = tT.shape
  CB = _CBREL
  NB = pl.cdiv(V, 4 * CB)

  def body(in_ref, out_ref):
    t = in_ref[...]
    stacked = jnp.concatenate(
        [t[:, m * CB:(m + 1) * CB] for m in range(4)], axis=0)
    tr = stacked.T
    lo = jnp.concatenate(
        [tr[:, m * D:m * D + D // 2] for m in range(4)], axis=1)
    hi = jnp.concatenate(
        [tr[:, m * D + D // 2:(m + 1) * D] for m in range(4)], axis=1)
    out_ref[...] = ((_rne_bf16_bits(lo) >> 16) & jnp.int32(0xFFFF)) | (
        _rne_bf16_bits(hi) & jnp.int32(-65536))

  return pl.pallas_call(
      body,
      grid=(NB,),
      in_specs=[pl.BlockSpec((D, 4 * CB), lambda i: (0, i))],
      out_specs=pl.BlockSpec((CB, 2 * D), lambda i: (i, 0)),
      out_shape=jax.ShapeDtypeStruct((NB * CB, 2 * D), jnp.int32),
  )(tT)


def _tc_head(x, pooled_sum, fc_w, fc_b2):
  B, S = x.shape
  D = pooled_sum.shape[1]
  C = fc_w.shape[0]
  BLK = 256
  assert B % BLK == 0

  def body(x_ref, ps_ref, w_ref, b_ref, out_ref):
    cnt = jnp.sum((x_ref[...] != 0).astype(jnp.float32), axis=1,
                  keepdims=True)
    pooled = jnp.maximum(ps_ref[...] / cnt, 0.0)
    out_ref[...] = lax.dot_general(
        w_ref[...], pooled, (((1,), (1,)), ((), ())),
        preferred_element_type=jnp.float32) + b_ref[...]

  return pl.pallas_call(
      body,
      grid=(B // BLK,),
      in_specs=[
          pl.BlockSpec((BLK, S), lambda i: (i, 0)),
          pl.BlockSpec((BLK, D), lambda i: (i, 0)),
          pl.BlockSpec((C, D), lambda i: (0, 0)),
          pl.BlockSpec((C, 1), lambda i: (0, 0)),
      ],
      out_specs=pl.BlockSpec((C, BLK), lambda i: (0, i)),
      out_shape=jax.ShapeDtypeStruct((C, B), jnp.float32),
  )(x, pooled_sum, fc_w, fc_b2)


def kernel(x, table, fc_w, fc_b):
  B, S = x.shape
  V, D = table.shape
  x = x.astype(jnp.int32)
  shift = _CBREL.bit_length() - 1
  q, r = x >> shift, x & (_CBREL - 1)
  px = (((q >> 2) << shift) + r) * 4 + (q & 3)
  x2 = px.reshape(2 * B, S // 2)
  tableT, x2 = jax.lax.optimization_barrier((table.T, x2))
  tableW = _tc_relayout(tableT)
  tableI = tableW.reshape(tableW.shape[0] * 4, D // 2)
  pooled_sum = _sc_pooled_sum(x2, tableI)
  return _tc_head(x, pooled_sum, fc_w, fc_b.reshape(-1, 1)).T


# speedup vs baseline: 1.0311x; 1.0311x over previous
import functools

import jax
import jax.numpy as jnp
from jax import lax
from jax.experimental import pallas as pl
from jax.experimental.pallas import tpu as pltpu
from jax.experimental.pallas import tpu_sc as plsc

_NC = 2
_NS = 16
_NW = _NC * _NS
_L = 16
_CHUNK = 100
_NBUF = 4
_CBREL = 4096


def _sc_pooled_sum(x2, table):
  twoB, chunk = x2.shape
  assert chunk == _CHUNK
  B = twoB // 2
  D = 2 * table.shape[1]
  assert D % (2 * _L) == 0 and B % _NW == 0
  b_per_w = B // _NW
  c_per_w = 2 * b_per_w
  nd = D // _L

  mesh = plsc.VectorSubcoreMesh(
      core_axis_name="c", subcore_axis_name="s",
      num_cores=_NC, num_subcores=_NS)

  @functools.partial(
      pl.kernel,
      out_type=jax.ShapeDtypeStruct((B, D), jnp.float32),
      mesh=mesh,
      scratch_types=[
          pltpu.VMEM((c_per_w, _CHUNK), jnp.int32),
          pltpu.VMEM((_NBUF, _CHUNK, D // 2), jnp.int32),
          pltpu.VMEM((b_per_w, D), jnp.float32),
      ] + [pltpu.SemaphoreType.DMA] * _NBUF,
      compiler_params=pltpu.CompilerParams(use_tc_tiling_on_sc=False),
  )
  def k(x2_hbm, table_hbm, out_hbm, idx_v, rows_v, out_v, *sems):
    wid = lax.axis_index("s") * _NC + lax.axis_index("c")
    cbase = wid * c_per_w
    bbase = wid * b_per_w

    pltpu.sync_copy(x2_hbm.at[pl.ds(cbase, c_per_w)], idx_v)

    for k0 in range(_NBUF):
      pltpu.async_copy(table_hbm.at[idx_v.at[k0]], rows_v.at[k0], sems[k0])

    def accum_chunk(buf, accs):
      def body(r4, accs):
        accs = list(accs)
        for u in range(4):
          r = r4 * 4 + u
          for h in range(nd // 2):
            w = rows_v[buf, r, h * _L:(h + 1) * _L]
            a = lax.bitcast_convert_type(w << 16, jnp.float32)
            b = lax.bitcast_convert_type(
                w & jnp.int32(-65536), jnp.float32)
            accs[h] = accs[h] + a
            accs[h + 2] = accs[h + 2] + b
        return tuple(accs)
      return lax.fori_loop(0, _CHUNK // 4, body, accs)

    def pair_body(p, carry):
      for half in range(2):
        i = 2 * p + half
        accs = tuple(jnp.zeros((_L,), jnp.float32) for _ in range(nd))
        for k1 in (2 * half, 2 * half + 1):
          c = 4 * p + k1
          pltpu.make_async_copy(
              table_hbm.at[idx_v.at[c]], rows_v.at[k1], sems[k1]).wait()
          accs = accum_chunk(k1, accs)

          @pl.when(c + _NBUF < c_per_w)
          def _():
            pltpu.async_copy(
                table_hbm.at[idx_v.at[c + _NBUF]], rows_v.at[k1], sems[k1])

        for d in range(nd):
          out_v[i, d * _L:(d + 1) * _L] = accs[d]
      return carry

    lax.fori_loop(0, b_per_w // 2, pair_body, 0)
    pltpu.sync_copy(out_v, out_hbm.at[pl.ds(bbase, b_per_w)])

  return k(x2, table)


def _rne_bf16_bits(x):
  bits = lax.bitcast_convert_type(x, jnp.int32)
  return bits + jnp.int32(0x7FFF) + ((bits >> 16) & jnp.int32(1))


def _tc_relayout(tT):
  D, V = tT.shape
  CB = _CBREL
  NB = pl.cdiv(V, 4 * CB)

  def body(in_ref, out_ref):
    t = in_ref[...]
    stacked = jnp.concatenate(
        [t[:, m * CB:(m + 1) * CB] for m in range(4)], axis=0)
    tr = stacked.T
    lo = jnp.concatenate(
        [tr[:, m * D:m * D + D // 2] for m in range(4)], axis=1)
    hi = jnp.concatenate(
        [tr[:, m * D + D // 2:(m + 1) * D] for m in range(4)], axis=1)
    out_ref[...] = ((_rne_bf16_bits(lo) >> 16) & jnp.int32(0xFFFF)) | (
        _rne_bf16_bits(hi) & jnp.int32(-65536))

  return pl.pallas_call(
      body,
      grid=(NB,),
      in_specs=[pl.BlockSpec((D, 4 * CB), lambda i: (0, i))],
      out_specs=pl.BlockSpec((CB, 2 * D), lambda i: (i, 0)),
      out_shape=jax.ShapeDtypeStruct((NB * CB, 2 * D), jnp.int32),
      compiler_params=pltpu.CompilerParams(
          vmem_limit_bytes=100 * 1024 * 1024),
  )(tT)


def _tc_head(x, pooled_sum, fc_w, fc_b2):
  B, S = x.shape
  D = pooled_sum.shape[1]
  C = fc_w.shape[0]
  BLK = 256
  assert B % BLK == 0

  def body(x_ref, ps_ref, w_ref, b_ref, out_ref):
    cnt = jnp.sum((x_ref[...] != 0).astype(jnp.float32), axis=1,
                  keepdims=True)
    pooled = jnp.maximum(ps_ref[...] / cnt, 0.0)
    out_ref[...] = lax.dot_general(
        w_ref[...], pooled, (((1,), (1,)), ((), ())),
        preferred_element_type=jnp.float32) + b_ref[...]

  return pl.pallas_call(
      body,
      grid=(B // BLK,),
      in_specs=[
          pl.BlockSpec((BLK, S), lambda i: (i, 0)),
          pl.BlockSpec((BLK, D), lambda i: (i, 0)),
          pl.BlockSpec((C, D), lambda i: (0, 0)),
          pl.BlockSpec((C, 1), lambda i: (0, 0)),
      ],
      out_specs=pl.BlockSpec((C, BLK), lambda i: (0, i)),
      out_shape=jax.ShapeDtypeStruct((C, B), jnp.float32),
  )(x, pooled_sum, fc_w, fc_b2)


def kernel(x, table, fc_w, fc_b):
  B, S = x.shape
  V, D = table.shape
  x = x.astype(jnp.int32)
  shift = _CBREL.bit_length() - 1
  q, r = x >> shift, x & (_CBREL - 1)
  px = (((q >> 2) << shift) + r) * 4 + (q & 3)
  x2 = px.reshape(2 * B, S // 2)
  tableT, x2 = jax.lax.optimization_barrier((table.T, x2))
  tableW = _tc_relayout(tableT)
  tableI = tableW.reshape(tableW.shape[0] * 4, D // 2)
  pooled_sum = _sc_pooled_sum(x2, tableI)
  return _tc_head(x, pooled_sum, fc_w, fc_b.reshape(-1, 1)).T


# speedup vs baseline: 1.5015x; 1.4563x over previous
import functools

import jax
import jax.numpy as jnp
from jax import lax
from jax.experimental import pallas as pl
from jax.experimental.pallas import tpu as pltpu
from jax.experimental.pallas import tpu_sc as plsc

_NC = 2
_NS = 16
_NW = _NC * _NS
_L = 16
_CHUNK = 100
_NBUF = 4
_CBREL = 4096


def _sc_pooled_sum(x2, table):
  twoB, chunk = x2.shape
  assert chunk == _CHUNK
  B = twoB // 2
  D = 2 * table.shape[1]
  assert D % (2 * _L) == 0 and B % _NW == 0
  b_per_w = B // _NW
  c_per_w = 2 * b_per_w
  nd = D // _L

  mesh = plsc.VectorSubcoreMesh(
      core_axis_name="c", subcore_axis_name="s",
      num_cores=_NC, num_subcores=_NS)

  @functools.partial(
      pl.kernel,
      out_type=jax.ShapeDtypeStruct((B, D), jnp.float32),
      mesh=mesh,
      scratch_types=[
          pltpu.VMEM((c_per_w, _CHUNK), jnp.int32),
          pltpu.VMEM((_NBUF, _CHUNK, D // 2), jnp.int32),
          pltpu.VMEM((b_per_w, D), jnp.float32),
      ] + [pltpu.SemaphoreType.DMA] * _NBUF,
      compiler_params=pltpu.CompilerParams(use_tc_tiling_on_sc=False),
  )
  def k(x2_hbm, table_hbm, out_hbm, idx_v, rows_v, out_v, *sems):
    wid = lax.axis_index("s") * _NC + lax.axis_index("c")
    cbase = wid * c_per_w
    bbase = wid * b_per_w

    pltpu.sync_copy(x2_hbm.at[pl.ds(cbase, c_per_w)], idx_v)

    for k0 in range(_NBUF):
      pltpu.async_copy(table_hbm.at[idx_v.at[k0]], rows_v.at[k0], sems[k0])

    def accum_chunk(buf, accs):
      def body(r4, accs):
        accs = list(accs)
        for u in range(4):
          r = r4 * 4 + u
          for h in range(nd // 2):
            w = rows_v[buf, r, h * _L:(h + 1) * _L]
            a = lax.bitcast_convert_type(w << 16, jnp.float32)
            b = lax.bitcast_convert_type(
                w & jnp.int32(-65536), jnp.float32)
            accs[h] = accs[h] + a
            accs[h + 2] = accs[h + 2] + b
        return tuple(accs)
      return lax.fori_loop(0, _CHUNK // 4, body, accs)

    def pair_body(p, carry):
      for half in range(2):
        i = 2 * p + half
        accs = tuple(jnp.zeros((_L,), jnp.float32) for _ in range(nd))
        for k1 in (2 * half, 2 * half + 1):
          c = 4 * p + k1
          pltpu.make_async_copy(
              table_hbm.at[idx_v.at[c]], rows_v.at[k1], sems[k1]).wait()
          accs = accum_chunk(k1, accs)

          @pl.when(c + _NBUF < c_per_w)
          def _():
            pltpu.async_copy(
                table_hbm.at[idx_v.at[c + _NBUF]], rows_v.at[k1], sems[k1])

        for d in range(nd):
          out_v[i, d * _L:(d + 1) * _L] = accs[d]
      return carry

    lax.fori_loop(0, b_per_w // 2, pair_body, 0)
    pltpu.sync_copy(out_v, out_hbm.at[pl.ds(bbase, b_per_w)])

  return k(x2, table)


def _rne_bf16_bits(x):
  bits = lax.bitcast_convert_type(x, jnp.int32)
  return bits + jnp.int32(0x7FFF) + ((bits >> 16) & jnp.int32(1))


def _tc_relayout(tT):
  D, V = tT.shape
  CB = _CBREL
  NB = pl.cdiv(V, 4 * CB)

  def body(in_ref, out_ref):
    t = in_ref[...]
    parts = []
    for m in range(4):
      tm = t[:, m * CB:(m + 1) * CB]
      parts.append(
          ((_rne_bf16_bits(tm[:D // 2]) >> 16) & jnp.int32(0xFFFF)) | (
              _rne_bf16_bits(tm[D // 2:]) & jnp.int32(-65536)))
    z = jnp.concatenate(parts, axis=0)
    out_ref[...] = z.T

  return pl.pallas_call(
      body,
      grid=(NB,),
      in_specs=[pl.BlockSpec((D, 4 * CB), lambda i: (0, i))],
      out_specs=pl.BlockSpec((CB, 2 * D), lambda i: (i, 0)),
      out_shape=jax.ShapeDtypeStruct((NB * CB, 2 * D), jnp.int32),
      compiler_params=pltpu.CompilerParams(
          vmem_limit_bytes=100 * 1024 * 1024),
  )(tT)


def _tc_head(x, pooled_sum, fc_w, fc_b2):
  B, S = x.shape
  D = pooled_sum.shape[1]
  C = fc_w.shape[0]
  BLK = 256
  assert B % BLK == 0

  def body(x_ref, ps_ref, w_ref, b_ref, out_ref):
    cnt = jnp.sum((x_ref[...] != 0).astype(jnp.float32), axis=1,
                  keepdims=True)
    pooled = jnp.maximum(ps_ref[...] / cnt, 0.0)
    out_ref[...] = lax.dot_general(
        w_ref[...], pooled, (((1,), (1,)), ((), ())),
        preferred_element_type=jnp.float32) + b_ref[...]

  return pl.pallas_call(
      body,
      grid=(B // BLK,),
      in_specs=[
          pl.BlockSpec((BLK, S), lambda i: (i, 0)),
          pl.BlockSpec((BLK, D), lambda i: (i, 0)),
          pl.BlockSpec((C, D), lambda i: (0, 0)),
          pl.BlockSpec((C, 1), lambda i: (0, 0)),
      ],
      out_specs=pl.BlockSpec((C, BLK), lambda i: (0, i)),
      out_shape=jax.ShapeDtypeStruct((C, B), jnp.float32),
  )(x, pooled_sum, fc_w, fc_b2)


def kernel(x, table, fc_w, fc_b):
  B, S = x.shape
  V, D = table.shape
  x = x.astype(jnp.int32)
  shift = _CBREL.bit_length() - 1
  q, r = x >> shift, x & (_CBREL - 1)
  px = (((q >> 2) << shift) + r) * 4 + (q & 3)
  x2 = px.reshape(2 * B, S // 2)
  tableT, x2 = jax.lax.optimization_barrier((table.T, x2))
  tableW = _tc_relayout(tableT)
  tableI = tableW.reshape(tableW.shape[0] * 4, D // 2)
  pooled_sum = _sc_pooled_sum(x2, tableI)
  return _tc_head(x, pooled_sum, fc_w, fc_b.reshape(-1, 1)).T


# speedup vs baseline: 1.5343x; 1.0218x over previous
import functools

import jax
import jax.numpy as jnp
from jax import lax
from jax.experimental import pallas as pl
from jax.experimental.pallas import tpu as pltpu
from jax.experimental.pallas import tpu_sc as plsc

_NC = 2
_NS = 16
_NW = _NC * _NS
_L = 16
_CHUNK = 100
_NBUF = 4
_CBREL = 8192


def _sc_pooled_sum(x2, table):
  twoB, chunk = x2.shape
  assert chunk == _CHUNK
  B = twoB // 2
  D = 2 * table.shape[1]
  assert D % (2 * _L) == 0 and B % _NW == 0
  b_per_w = B // _NW
  c_per_w = 2 * b_per_w
  nd = D // _L

  mesh = plsc.VectorSubcoreMesh(
      core_axis_name="c", subcore_axis_name="s",
      num_cores=_NC, num_subcores=_NS)

  @functools.partial(
      pl.kernel,
      out_type=jax.ShapeDtypeStruct((B, D), jnp.float32),
      mesh=mesh,
      scratch_types=[
          pltpu.VMEM((c_per_w, _CHUNK), jnp.int32),
          pltpu.VMEM((_NBUF, _CHUNK, D // 2), jnp.int32),
          pltpu.VMEM((b_per_w, D), jnp.float32),
      ] + [pltpu.SemaphoreType.DMA] * _NBUF,
      compiler_params=pltpu.CompilerParams(use_tc_tiling_on_sc=False),
  )
  def k(x2_hbm, table_hbm, out_hbm, idx_v, rows_v, out_v, *sems):
    wid = lax.axis_index("s") * _NC + lax.axis_index("c")
    cbase = wid * c_per_w
    bbase = wid * b_per_w

    pltpu.sync_copy(x2_hbm.at[pl.ds(cbase, c_per_w)], idx_v)

    for k0 in range(_NBUF):
      pltpu.async_copy(table_hbm.at[idx_v.at[k0]], rows_v.at[k0], sems[k0])

    def accum_chunk(buf, accs):
      def body(r4, accs):
        accs = list(accs)
        for u in range(4):
          r = r4 * 4 + u
          for h in range(nd // 2):
            w = rows_v[buf, r, h * _L:(h + 1) * _L]
            a = lax.bitcast_convert_type(w << 16, jnp.float32)
            b = lax.bitcast_convert_type(
                w & jnp.int32(-65536), jnp.float32)
            accs[h] = accs[h] + a
            accs[h + 2] = accs[h + 2] + b
        return tuple(accs)
      return lax.fori_loop(0, _CHUNK // 4, body, accs)

    def pair_body(p, carry):
      for half in range(2):
        i = 2 * p + half
        accs = tuple(jnp.zeros((_L,), jnp.float32) for _ in range(nd))
        for k1 in (2 * half, 2 * half + 1):
          c = 4 * p + k1
          pltpu.make_async_copy(
              table_hbm.at[idx_v.at[c]], rows_v.at[k1], sems[k1]).wait()
          accs = accum_chunk(k1, accs)

          @pl.when(c + _NBUF < c_per_w)
          def _():
            pltpu.async_copy(
                table_hbm.at[idx_v.at[c + _NBUF]], rows_v.at[k1], sems[k1])

        for d in range(nd):
          out_v[i, d * _L:(d + 1) * _L] = accs[d]
      return carry

    lax.fori_loop(0, b_per_w // 2, pair_body, 0)
    pltpu.sync_copy(out_v, out_hbm.at[pl.ds(bbase, b_per_w)])

  return k(x2, table)


def _rne_bf16_bits(x):
  bits = lax.bitcast_convert_type(x, jnp.int32)
  return bits + jnp.int32(0x7FFF) + ((bits >> 16) & jnp.int32(1))


def _tc_relayout(tT):
  D, V = tT.shape
  CB = _CBREL
  NB = pl.cdiv(V, 4 * CB)

  def body(in_ref, out_ref):
    t = in_ref[...]
    parts = []
    for m in range(4):
      tm = t[:, m * CB:(m + 1) * CB]
      parts.append(
          ((_rne_bf16_bits(tm[:D // 2]) >> 16) & jnp.int32(0xFFFF)) | (
              _rne_bf16_bits(tm[D // 2:]) & jnp.int32(-65536)))
    z = jnp.concatenate(parts, axis=0)
    out_ref[...] = z.T

  return pl.pallas_call(
      body,
      grid=(NB,),
      in_specs=[pl.BlockSpec((D, 4 * CB), lambda i: (0, i))],
      out_specs=pl.BlockSpec((CB, 2 * D), lambda i: (i, 0)),
      out_shape=jax.ShapeDtypeStruct((NB * CB, 2 * D), jnp.int32),
      compiler_params=pltpu.CompilerParams(
          vmem_limit_bytes=100 * 1024 * 1024),
  )(tT)


def _tc_head(x, pooled_sum, fc_w, fc_b2):
  B, S = x.shape
  D = pooled_sum.shape[1]
  C = fc_w.shape[0]
  BLK = 256
  assert B % BLK == 0

  def body(x_ref, ps_ref, w_ref, b_ref, out_ref):
    cnt = jnp.sum((x_ref[...] != 0).astype(jnp.float32), axis=1,
                  keepdims=True)
    pooled = jnp.maximum(ps_ref[...] / cnt, 0.0)
    out_ref[...] = lax.dot_general(
        w_ref[...], pooled, (((1,), (1,)), ((), ())),
        preferred_element_type=jnp.float32) + b_ref[...]

  return pl.pallas_call(
      body,
      grid=(B // BLK,),
      in_specs=[
          pl.BlockSpec((BLK, S), lambda i: (i, 0)),
          pl.BlockSpec((BLK, D), lambda i: (i, 0)),
          pl.BlockSpec((C, D), lambda i: (0, 0)),
          pl.BlockSpec((C, 1), lambda i: (0, 0)),
      ],
      out_specs=pl.BlockSpec((C, BLK), lambda i: (0, i)),
      out_shape=jax.ShapeDtypeStruct((C, B), jnp.float32),
  )(x, pooled_sum, fc_w, fc_b2)


def kernel(x, table, fc_w, fc_b):
  B, S = x.shape
  V, D = table.shape
  x = x.astype(jnp.int32)
  shift = _CBREL.bit_length() - 1
  q, r = x >> shift, x & (_CBREL - 1)
  px = (((q >> 2) << shift) + r) * 4 + (q & 3)
  x2 = px.reshape(2 * B, S // 2)
  tableT, x2 = jax.lax.optimization_barrier((table.T, x2))
  tableW = _tc_relayout(tableT)
  tableI = tableW.reshape(tableW.shape[0] * 4, D // 2)
  pooled_sum = _sc_pooled_sum(x2, tableI)
  return _tc_head(x, pooled_sum, fc_w, fc_b.reshape(-1, 1)).T
